# baseline (speedup 1.0000x reference)
import functools

import jax
import jax.numpy as jnp
from jax import lax
from jax.experimental import pallas as pl
from jax.experimental.pallas import tpu as pltpu
from jax.experimental.pallas import tpu_sc as plsc

NC = 2
NS = 16
NW = NC * NS
NSLC = 5


def _proj_body(uf, w1u, itf, a1i, ab1, rt, w1r, b1, upc, ipc, rpb):
  upc[...] = jnp.dot(uf[...], w1u[...], preferred_element_type=jnp.float32)
  ipc[...] = (
      jnp.dot(itf[...], a1i[...], preferred_element_type=jnp.float32)
      + ab1[...]
  )

  @pl.when(pl.program_id(0) == 0)
  def _():
    rpb[...] = (
        jnp.dot(rt[...], w1r[...], preferred_element_type=jnp.float32)
        + b1[...]
    )


def _edge_body(ug, ig, rat, rpb8, w2, b2, a1f, a2, ab2, a3, ab3, msg, eout):
  be = ug.shape[0]
  rrow = rat[0]
  oh = (lax.broadcasted_iota(jnp.int32, (8, be), 0) == rrow).astype(
      jnp.float32
  )
  radd = lax.dot_general(
      oh, rpb8[...], (((0,), (0,)), ((), ())),
      preferred_element_type=jnp.float32,
  )
  h = jnp.maximum(ug[...] + radd, 0.0)
  f = jnp.maximum(
      jnp.dot(h, w2[...], preferred_element_type=jnp.float32) + b2[...], 0.0
  )
  a = jnp.maximum(
      jnp.dot(f, a1f[...], preferred_element_type=jnp.float32) + ig[...], 0.0
  )
  a = jnp.maximum(
      jnp.dot(a, a2[...], preferred_element_type=jnp.float32) + ab2[...], 0.0
  )
  s = jnp.dot(a, a3[...], preferred_element_type=jnp.float32) + ab3[0, 0]
  e = jnp.exp(jnp.clip(s, -60.0, 60.0))
  msg[...] = f * e
  eout[...] = jnp.swapaxes(e, 0, 1).reshape(1, 1, be)


def _final_body(num, den, ww, wb, out):
  acc = num[0]
  drow = den[0:1, :]
  for i in range(1, num.shape[0]):
    acc = acc + num[i]
    drow = drow + den[i : i + 1, :]
  dinv = jnp.where(drow > 0.0, 1.0 / drow, 0.0)
  dcol = jnp.swapaxes(dinv, 0, 1)
  out[...] = (
      jnp.dot(acc * dcol, ww[...], preferred_element_type=jnp.float32)
      + wb[...]
  )


def kernel(user_feat, item_feat, rating_table, row_idx, col_idx, rating,
           W1, b1, W2, b2, A1, ab1, A2, ab2, A3, ab3, Ww, wb):
  n_user, d = user_feat.shape
  n_item = item_feat.shape[0]
  n_rat = rating_table.shape[0]
  e_num = row_idx.shape[0]
  ck = 128
  n_chunks = e_num // ck
  ncs = n_chunks // NSLC
  es = e_num // NSLC
  npad = ((n_item + NS * 16 - 1) // (NS * 16)) * (NS * 16)
  rpt = npad // NS

  w1u = W1[:d]
  w1r = W1[d:]
  a1f = A1[:d]
  a1i = A1[d:]

  bn = 1000
  grid_n = n_user // bn
  upc, ipc, rpb = pl.pallas_call(
      _proj_body,
      grid=(grid_n,),
      in_specs=[
          pl.BlockSpec((bn, d), lambda i: (i, 0)),
          pl.BlockSpec((d, d), lambda i: (0, 0)),
          pl.BlockSpec((bn, d), lambda i: (i, 0)),
          pl.BlockSpec((d, d), lambda i: (0, 0)),
          pl.BlockSpec((1, d), lambda i: (0, 0)),
          pl.BlockSpec((n_rat, d), lambda i: (0, 0)),
          pl.BlockSpec((d, d), lambda i: (0, 0)),
          pl.BlockSpec((1, d), lambda i: (0, 0)),
      ],
      out_specs=[
          pl.BlockSpec((bn, d), lambda i: (i, 0)),
          pl.BlockSpec((bn, d), lambda i: (i, 0)),
          pl.BlockSpec((n_rat, d), lambda i: (0, 0)),
      ],
      out_shape=[
          jax.ShapeDtypeStruct((n_user, d), jnp.float32),
          jax.ShapeDtypeStruct((n_item, d), jnp.float32),
          jax.ShapeDtypeStruct((n_rat, d), jnp.float32),
      ],
      compiler_params=pltpu.CompilerParams(
          dimension_semantics=("arbitrary",)
      ),
  )(user_feat, w1u, item_feat, a1i, ab1.reshape(1, d), rating_table, w1r,
    b1.reshape(1, d))

  rpb8 = jnp.concatenate([rpb, jnp.zeros((8 - n_rat, d), jnp.float32)])
  mesh = plsc.VectorSubcoreMesh(core_axis_name="c", subcore_axis_name="s")

  kpt = 2 * ((ncs + 2 * NW - 1) // (2 * NW))
  npairs = kpt // 2
  padc = NW * kpt - ncs

  def _by_tile_g(x):
    xp = jnp.concatenate([x, jnp.broadcast_to(x[-1:], (padc, ck))])
    return xp.reshape(kpt, NW, ck).transpose(1, 0, 2)

  nstage = ((n_user + NS * 8 - 1) // (NS * 8)) * NS * 8
  spt = nstage // NS

  @functools.partial(
      pl.kernel,
      out_type=(
          jax.ShapeDtypeStruct((ncs, ck, d), jnp.float32),
          jax.ShapeDtypeStruct((ncs, ck, d), jnp.float32),
      ),
      mesh=mesh,
      scratch_types=[
          pltpu.VMEM((kpt, ck), jnp.int32),
          pltpu.VMEM((kpt, ck), jnp.int32),
          pltpu.VMEM((ck, d), jnp.float32),
          pltpu.VMEM((ck, d), jnp.float32),
          pltpu.VMEM((ck, d), jnp.float32),
          pltpu.VMEM((ck, d), jnp.float32),
          pltpu.VMEM_SHARED((nstage, d), jnp.float32),
          [pltpu.SemaphoreType.DMA] * 4,
          [pltpu.SemaphoreType.DMA] * 4,
      ],
  )
  def _gather(upc_h, ipc_h, row_h, col_h, ug_h, ig_h,
              ridx_v, cidx_v, ubufa, ibufa, ubufb, ibufb, spm, gsems, wsems):
    c = lax.axis_index("c")
    s = lax.axis_index("s")
    wid = s * NC + c
    pltpu.sync_copy(upc_h.at[pl.ds(s * spt, spt)], spm.at[pl.ds(s * spt, spt)])
    pltpu.sync_copy(row_h.at[wid], ridx_v)
    pltpu.sync_copy(col_h.at[wid], cidx_v)
    plsc.subcore_barrier()
    last = ncs - 1

    def body(m, carry):
      cida = jnp.minimum((2 * m) * NW + wid, last)
      cidb = jnp.minimum((2 * m + 1) * NW + wid, last)

      @pl.when(m > 0)
      def _():
        pltpu.make_async_copy(ubufa, ug_h.at[cida], wsems[0]).wait()
        pltpu.make_async_copy(ibufa, ig_h.at[cida], wsems[1]).wait()

      ga1 = pltpu.async_copy(spm.at[ridx_v.at[2 * m]], ubufa, gsems[0])
      ga2 = pltpu.async_copy(ipc_h.at[cidx_v.at[2 * m]], ibufa, gsems[1])

      @pl.when(m > 0)
      def _():
        pltpu.make_async_copy(ubufb, ug_h.at[cidb], wsems[2]).wait()
        pltpu.make_async_copy(ibufb, ig_h.at[cidb], wsems[3]).wait()

      gb1 = pltpu.async_copy(spm.at[ridx_v.at[2 * m + 1]], ubufb, gsems[2])
      gb2 = pltpu.async_copy(ipc_h.at[cidx_v.at[2 * m + 1]], ibufb, gsems[3])
      ga1.wait()
      ga2.wait()
      pltpu.make_async_copy(ubufa, ug_h.at[cida], wsems[0]).start()
      pltpu.make_async_copy(ibufa, ig_h.at[cida], wsems[1]).start()
      gb1.wait()
      gb2.wait()
      pltpu.make_async_copy(ubufb, ug_h.at[cidb], wsems[2]).start()
      pltpu.make_async_copy(ibufb, ig_h.at[cidb], wsems[3]).start()
      return carry

    lax.fori_loop(0, npairs, body, 0)
    pltpu.make_async_copy(ubufa, ug_h.at[0], wsems[0]).wait()
    pltpu.make_async_copy(ibufa, ig_h.at[0], wsems[1]).wait()
    pltpu.make_async_copy(ubufb, ug_h.at[0], wsems[2]).wait()
    pltpu.make_async_copy(ibufb, ig_h.at[0], wsems[3]).wait()

  be = 2000
  grid_e = es // be

  edge_call = pl.pallas_call(
      _edge_body,
      grid=(grid_e,),
      in_specs=[
          pl.BlockSpec((be, d), lambda i: (i, 0)),
          pl.BlockSpec((be, d), lambda i: (i, 0)),
          pl.BlockSpec((1, 1, be), lambda i: (i, 0, 0)),
          pl.BlockSpec((8, d), lambda i: (0, 0)),
          pl.BlockSpec((d, d), lambda i: (0, 0)),
          pl.BlockSpec((1, d), lambda i: (0, 0)),
          pl.BlockSpec((d, d), lambda i: (0, 0)),
          pl.BlockSpec((d, d), lambda i: (0, 0)),
          pl.BlockSpec((1, d), lambda i: (0, 0)),
          pl.BlockSpec((d, 1), lambda i: (0, 0)),
          pl.BlockSpec((1, 1), lambda i: (0, 0)),
      ],
      out_specs=[
          pl.BlockSpec((be, d), lambda i: (i, 0)),
          pl.BlockSpec((1, 1, be), lambda i: (i, 0, 0)),
      ],
      out_shape=[
          jax.ShapeDtypeStruct((es, d), jnp.float32),
          jax.ShapeDtypeStruct((grid_e, 1, be), jnp.float32),
      ],
      compiler_params=pltpu.CompilerParams(
          dimension_semantics=("arbitrary",)
      ),
  )

  cpc = ncs // NC
  kpt2 = 2 * ((cpc + 2 * NS - 1) // (2 * NS))
  padc2 = NS * kpt2 - cpc

  def _by_tile_s(x):
    xc = x.reshape(NC, cpc, ck)
    xc = jnp.concatenate(
        [xc, jnp.broadcast_to(xc[:, -1:], (NC, padc2, ck))], axis=1
    )
    return xc.reshape(NC, kpt2, NS, ck).transpose(0, 2, 1, 3).reshape(
        NC * NS, kpt2, ck
    )

  zn = jnp.zeros((rpt, d), jnp.float32)
  zd = jnp.zeros((rpt,), jnp.float32)

  @functools.partial(
      pl.kernel,
      out_type=(
          jax.ShapeDtypeStruct((NC * npad, d), jnp.float32),
          jax.ShapeDtypeStruct((NC * npad,), jnp.float32),
      ),
      mesh=mesh,
      scratch_types=[
          pltpu.VMEM((kpt2, ck), jnp.int32),
          pltpu.VMEM((kpt2, ck), jnp.float32),
          pltpu.VMEM((ck, d), jnp.float32),
          pltpu.VMEM((ck, d), jnp.float32),
          pltpu.VMEM_SHARED((npad, d), jnp.float32),
          pltpu.VMEM_SHARED((npad,), jnp.float32),
          [pltpu.SemaphoreType.DMA] * 4,
      ],
      compiler_params=pltpu.CompilerParams(use_tc_tiling_on_sc=False),
  )
  def _scatter(msg_h, e_h, col_h, zn_h, zd_h, num_h, den_h,
               cidx_v, ev_v, vala, valb, accn, accd, sems):
    c = lax.axis_index("c")
    s = lax.axis_index("s")
    widc = c * NS + s
    pltpu.sync_copy(col_h.at[widc], cidx_v)
    pltpu.sync_copy(e_h.at[widc], ev_v)
    pltpu.sync_copy(zn_h, accn.at[pl.ds(s * rpt, rpt)])
    pltpu.sync_copy(zd_h, accd.at[pl.ds(s * rpt, rpt)])
    plsc.subcore_barrier()

    def chunk(m, slot, val_v):
      k = 2 * m + slot
      j = k * NS + s

      @pl.when((m > 0) & (j - 2 * NS < cpc))
      def _():
        kp = k - 2
        pltpu.make_async_copy(
            val_v, accn.at[cidx_v.at[kp]], sems[2 * slot]
        ).wait()
        pltpu.make_async_copy(
            ev_v.at[kp], accd.at[cidx_v.at[kp]], sems[2 * slot + 1]
        ).wait()

      @pl.when(j < cpc)
      def _():
        cid = c * cpc + j
        pltpu.sync_copy(msg_h.at[pl.ds(cid * ck, ck)], val_v)
        pltpu.make_async_copy(
            val_v, accn.at[cidx_v.at[k]], sems[2 * slot]
        ).start(add=True)
        pltpu.make_async_copy(
            ev_v.at[k], accd.at[cidx_v.at[k]], sems[2 * slot + 1]
        ).start(add=True)

    def body(m, carry):
      chunk(m, 0, vala)
      chunk(m, 1, valb)
      return carry

    lax.fori_loop(0, kpt2 // 2, body, 0)
    for slot in (0, 1):
      k = kpt2 - 2 + slot

      @pl.when(k * NS + s < cpc)
      def _():
        pltpu.make_async_copy(
            vala if slot == 0 else valb,
            accn.at[cidx_v.at[k]], sems[2 * slot],
        ).wait()
        pltpu.make_async_copy(
            ev_v.at[k], accd.at[cidx_v.at[k]], sems[2 * slot + 1]
        ).wait()

    plsc.subcore_barrier()
    pltpu.sync_copy(
        accn.at[pl.ds(s * rpt, rpt)],
        num_h.at[pl.ds(c * npad + s * rpt, rpt)],
    )
    pltpu.sync_copy(
        accd.at[pl.ds(s * rpt, rpt)],
        den_h.at[pl.ds(c * npad + s * rpt, rpt)],
    )

  row2 = row_idx.reshape(n_chunks, ck)
  col2 = col_idx.reshape(n_chunks, ck)
  rat3 = rating.reshape(e_num // be, 1, be)
  pad_rows = jnp.zeros((nstage - n_user, d), jnp.float32)
  upcp = jnp.concatenate([upc, pad_rows])
  ipcp = jnp.concatenate([ipc, pad_rows])
  nums, dens = [], []
  for h in range(NSLC):
    rows_s = row2[h * ncs : (h + 1) * ncs]
    cols_s = col2[h * ncs : (h + 1) * ncs]
    ug3, ig3 = _gather(upcp, ipcp, _by_tile_g(rows_s), _by_tile_g(cols_s))
    msg, erows = edge_call(
        ug3.reshape(es, d), ig3.reshape(es, d),
        rat3[h * grid_e : (h + 1) * grid_e], rpb8, W2, b2.reshape(1, d),
        a1f, A2, ab2.reshape(1, d), A3, ab3.reshape(1, 1),
    )
    numf, denf = _scatter(
        msg, _by_tile_s(erows.reshape(ncs, ck)), _by_tile_s(cols_s), zn, zd
    )
    nums.append(numf.reshape(NC, npad, d))
    dens.append(denf.reshape(NC, npad))

  num = jnp.concatenate(nums)
  den = jnp.concatenate(dens)
  nparts = NC * NSLC

  bz = 1024
  grid_z = npad // bz
  z = pl.pallas_call(
      _final_body,
      grid=(grid_z,),
      in_specs=[
          pl.BlockSpec((nparts, bz, d), lambda i: (0, i, 0)),
          pl.BlockSpec((nparts, bz), lambda i: (0, i)),
          pl.BlockSpec((d, d), lambda i: (0, 0)),
          pl.BlockSpec((1, d), lambda i: (0, 0)),
      ],
      out_specs=pl.BlockSpec((bz, d), lambda i: (i, 0)),
      out_shape=jax.ShapeDtypeStruct((npad, d), jnp.float32),
      compiler_params=pltpu.CompilerParams(
          dimension_semantics=("arbitrary",)
      ),
  )(num, den, Ww, wb.reshape(1, d))
  return z[:n_item]


# speedup vs baseline: 1.0036x; 1.0036x over previous
import functools

import jax
import jax.numpy as jnp
from jax import lax
from jax.experimental import pallas as pl
from jax.experimental.pallas import tpu as pltpu
from jax.experimental.pallas import tpu_sc as plsc

NC = 2
NS = 16
NW = NC * NS
NSLC = 2


def _proj_body(uf, w1u, itf, a1i, ab1, rt, w1r, b1, upc, ipc, rpb):
  upc[...] = jnp.dot(uf[...], w1u[...], preferred_element_type=jnp.float32)
  ipc[...] = (
      jnp.dot(itf[...], a1i[...], preferred_element_type=jnp.float32)
      + ab1[...]
  )

  @pl.when(pl.program_id(0) == 0)
  def _():
    rpb[...] = (
        jnp.dot(rt[...], w1r[...], preferred_element_type=jnp.float32)
        + b1[...]
    )


def _edge_body(ug, ig, rat, rpb8, w2, b2, a1f, a2, ab2, a3, ab3, msg, eout):
  be = ug.shape[0]
  rrow = rat[0]
  oh = (lax.broadcasted_iota(jnp.int32, (8, be), 0) == rrow).astype(
      jnp.float32
  )
  radd = lax.dot_general(
      oh, rpb8[...], (((0,), (0,)), ((), ())),
      preferred_element_type=jnp.float32,
  )
  h = jnp.maximum(ug[...] + radd, 0.0)
  f = jnp.maximum(
      jnp.dot(h, w2[...], preferred_element_type=jnp.float32) + b2[...], 0.0
  )
  a = jnp.maximum(
      jnp.dot(f, a1f[...], preferred_element_type=jnp.float32) + ig[...], 0.0
  )
  a = jnp.maximum(
      jnp.dot(a, a2[...], preferred_element_type=jnp.float32) + ab2[...], 0.0
  )
  s = jnp.dot(a, a3[...], preferred_element_type=jnp.float32) + ab3[0, 0]
  e = jnp.exp(jnp.clip(s, -60.0, 60.0))
  msg[...] = f * e
  eout[...] = jnp.swapaxes(e, 0, 1).reshape(1, 1, be)


def _final_body(num, den, ww, wb, out):
  acc = num[0]
  drow = den[0:1, :]
  for i in range(1, num.shape[0]):
    acc = acc + num[i]
    drow = drow + den[i : i + 1, :]
  dinv = jnp.where(drow > 0.0, 1.0 / drow, 0.0)
  dcol = jnp.swapaxes(dinv, 0, 1)
  out[...] = (
      jnp.dot(acc * dcol, ww[...], preferred_element_type=jnp.float32)
      + wb[...]
  )


def kernel(user_feat, item_feat, rating_table, row_idx, col_idx, rating,
           W1, b1, W2, b2, A1, ab1, A2, ab2, A3, ab3, Ww, wb):
  n_user, d = user_feat.shape
  n_item = item_feat.shape[0]
  n_rat = rating_table.shape[0]
  e_num = row_idx.shape[0]
  ck = 128
  n_chunks = e_num // ck
  ncs = n_chunks // NSLC
  es = e_num // NSLC
  npad = ((n_item + NS * 16 - 1) // (NS * 16)) * (NS * 16)
  rpt = npad // NS

  w1u = W1[:d]
  w1r = W1[d:]
  a1f = A1[:d]
  a1i = A1[d:]

  bn = 1000
  grid_n = n_user // bn
  upc, ipc, rpb = pl.pallas_call(
      _proj_body,
      grid=(grid_n,),
      in_specs=[
          pl.BlockSpec((bn, d), lambda i: (i, 0)),
          pl.BlockSpec((d, d), lambda i: (0, 0)),
          pl.BlockSpec((bn, d), lambda i: (i, 0)),
          pl.BlockSpec((d, d), lambda i: (0, 0)),
          pl.BlockSpec((1, d), lambda i: (0, 0)),
          pl.BlockSpec((n_rat, d), lambda i: (0, 0)),
          pl.BlockSpec((d, d), lambda i: (0, 0)),
          pl.BlockSpec((1, d), lambda i: (0, 0)),
      ],
      out_specs=[
          pl.BlockSpec((bn, d), lambda i: (i, 0)),
          pl.BlockSpec((bn, d), lambda i: (i, 0)),
          pl.BlockSpec((n_rat, d), lambda i: (0, 0)),
      ],
      out_shape=[
          jax.ShapeDtypeStruct((n_user, d), jnp.float32),
          jax.ShapeDtypeStruct((n_item, d), jnp.float32),
          jax.ShapeDtypeStruct((n_rat, d), jnp.float32),
      ],
      compiler_params=pltpu.CompilerParams(
          dimension_semantics=("arbitrary",)
      ),
  )(user_feat, w1u, item_feat, a1i, ab1.reshape(1, d), rating_table, w1r,
    b1.reshape(1, d))

  rpb8 = jnp.concatenate([rpb, jnp.zeros((8 - n_rat, d), jnp.float32)])
  mesh = plsc.VectorSubcoreMesh(core_axis_name="c", subcore_axis_name="s")

  kpt = 2 * ((ncs + 2 * NW - 1) // (2 * NW))
  npairs = kpt // 2
  padc = NW * kpt - ncs

  def _by_tile_g(x):
    xp = jnp.concatenate([x, jnp.broadcast_to(x[-1:], (padc, ck))])
    return xp.reshape(kpt, NW, ck).transpose(1, 0, 2)

  nstage = ((n_user + NS * 8 - 1) // (NS * 8)) * NS * 8
  spt = nstage // NS

  @functools.partial(
      pl.kernel,
      out_type=(
          jax.ShapeDtypeStruct((ncs, ck, d), jnp.float32),
          jax.ShapeDtypeStruct((ncs, ck, d), jnp.float32),
      ),
      mesh=mesh,
      scratch_types=[
          pltpu.VMEM((kpt, ck), jnp.int32),
          pltpu.VMEM((kpt, ck), jnp.int32),
          pltpu.VMEM((ck, d), jnp.float32),
          pltpu.VMEM((ck, d), jnp.float32),
          pltpu.VMEM((ck, d), jnp.float32),
          pltpu.VMEM((ck, d), jnp.float32),
          pltpu.VMEM_SHARED((nstage, d), jnp.float32),
          [pltpu.SemaphoreType.DMA] * 4,
          [pltpu.SemaphoreType.DMA] * 4,
      ],
  )
  def _gather(upc_h, ipc_h, row_h, col_h, ug_h, ig_h,
              ridx_v, cidx_v, ubufa, ibufa, ubufb, ibufb, spm, gsems, wsems):
    c = lax.axis_index("c")
    s = lax.axis_index("s")
    wid = s * NC + c
    pltpu.sync_copy(upc_h.at[pl.ds(s * spt, spt)], spm.at[pl.ds(s * spt, spt)])
    pltpu.sync_copy(row_h.at[wid], ridx_v)
    pltpu.sync_copy(col_h.at[wid], cidx_v)
    plsc.subcore_barrier()
    last = ncs - 1

    def body(m, carry):
      cida = jnp.minimum((2 * m) * NW + wid, last)
      cidb = jnp.minimum((2 * m + 1) * NW + wid, last)

      @pl.when(m > 0)
      def _():
        pltpu.make_async_copy(ubufa, ug_h.at[cida], wsems[0]).wait()
        pltpu.make_async_copy(ibufa, ig_h.at[cida], wsems[1]).wait()

      ga1 = pltpu.async_copy(spm.at[ridx_v.at[2 * m]], ubufa, gsems[0])
      ga2 = pltpu.async_copy(ipc_h.at[cidx_v.at[2 * m]], ibufa, gsems[1])

      @pl.when(m > 0)
      def _():
        pltpu.make_async_copy(ubufb, ug_h.at[cidb], wsems[2]).wait()
        pltpu.make_async_copy(ibufb, ig_h.at[cidb], wsems[3]).wait()

      gb1 = pltpu.async_copy(spm.at[ridx_v.at[2 * m + 1]], ubufb, gsems[2])
      gb2 = pltpu.async_copy(ipc_h.at[cidx_v.at[2 * m + 1]], ibufb, gsems[3])
      ga1.wait()
      ga2.wait()
      pltpu.make_async_copy(ubufa, ug_h.at[cida], wsems[0]).start()
      pltpu.make_async_copy(ibufa, ig_h.at[cida], wsems[1]).start()
      gb1.wait()
      gb2.wait()
      pltpu.make_async_copy(ubufb, ug_h.at[cidb], wsems[2]).start()
      pltpu.make_async_copy(ibufb, ig_h.at[cidb], wsems[3]).start()
      return carry

    lax.fori_loop(0, npairs, body, 0)
    pltpu.make_async_copy(ubufa, ug_h.at[0], wsems[0]).wait()
    pltpu.make_async_copy(ibufa, ig_h.at[0], wsems[1]).wait()
    pltpu.make_async_copy(ubufb, ug_h.at[0], wsems[2]).wait()
    pltpu.make_async_copy(ibufb, ig_h.at[0], wsems[3]).wait()

  be = 2000
  grid_e = es // be

  edge_call = pl.pallas_call(
      _edge_body,
      grid=(grid_e,),
      in_specs=[
          pl.BlockSpec((be, d), lambda i: (i, 0)),
          pl.BlockSpec((be, d), lambda i: (i, 0)),
          pl.BlockSpec((1, 1, be), lambda i: (i, 0, 0)),
          pl.BlockSpec((8, d), lambda i: (0, 0)),
          pl.BlockSpec((d, d), lambda i: (0, 0)),
          pl.BlockSpec((1, d), lambda i: (0, 0)),
          pl.BlockSpec((d, d), lambda i: (0, 0)),
          pl.BlockSpec((d, d), lambda i: (0, 0)),
          pl.BlockSpec((1, d), lambda i: (0, 0)),
          pl.BlockSpec((d, 1), lambda i: (0, 0)),
          pl.BlockSpec((1, 1), lambda i: (0, 0)),
      ],
      out_specs=[
          pl.BlockSpec((be, d), lambda i: (i, 0)),
          pl.BlockSpec((1, 1, be), lambda i: (i, 0, 0)),
      ],
      out_shape=[
          jax.ShapeDtypeStruct((es, d), jnp.float32),
          jax.ShapeDtypeStruct((grid_e, 1, be), jnp.float32),
      ],
      compiler_params=pltpu.CompilerParams(
          dimension_semantics=("arbitrary",)
      ),
  )

  cpc = ncs // NC
  kpt2 = 2 * ((cpc + 2 * NS - 1) // (2 * NS))
  padc2 = NS * kpt2 - cpc

  def _by_tile_s(x):
    xc = x.reshape(NC, cpc, ck)
    xc = jnp.concatenate(
        [xc, jnp.broadcast_to(xc[:, -1:], (NC, padc2, ck))], axis=1
    )
    return xc.reshape(NC, kpt2, NS, ck).transpose(0, 2, 1, 3).reshape(
        NC * NS, kpt2, ck
    )

  zn = jnp.zeros((rpt, d), jnp.float32)
  zd = jnp.zeros((rpt,), jnp.float32)

  @functools.partial(
      pl.kernel,
      out_type=(
          jax.ShapeDtypeStruct((NC * npad, d), jnp.float32),
          jax.ShapeDtypeStruct((NC * npad,), jnp.float32),
      ),
      mesh=mesh,
      scratch_types=[
          pltpu.VMEM((kpt2, ck), jnp.int32),
          pltpu.VMEM((kpt2, ck), jnp.float32),
          pltpu.VMEM((ck, d), jnp.float32),
          pltpu.VMEM((ck, d), jnp.float32),
          pltpu.VMEM_SHARED((npad, d), jnp.float32),
          pltpu.VMEM_SHARED((npad,), jnp.float32),
          [pltpu.SemaphoreType.DMA] * 4,
      ],
      compiler_params=pltpu.CompilerParams(use_tc_tiling_on_sc=False),
  )
  def _scatter(msg_h, e_h, col_h, zn_h, zd_h, num_h, den_h,
               cidx_v, ev_v, vala, valb, accn, accd, sems):
    c = lax.axis_index("c")
    s = lax.axis_index("s")
    widc = c * NS + s
    pltpu.sync_copy(col_h.at[widc], cidx_v)
    pltpu.sync_copy(e_h.at[widc], ev_v)
    pltpu.sync_copy(zn_h, accn.at[pl.ds(s * rpt, rpt)])
    pltpu.sync_copy(zd_h, accd.at[pl.ds(s * rpt, rpt)])
    plsc.subcore_barrier()

    def chunk(m, slot, val_v):
      k = 2 * m + slot
      j = k * NS + s

      @pl.when((m > 0) & (j - 2 * NS < cpc))
      def _():
        kp = k - 2
        pltpu.make_async_copy(
            val_v, accn.at[cidx_v.at[kp]], sems[2 * slot]
        ).wait()
        pltpu.make_async_copy(
            ev_v.at[kp], accd.at[cidx_v.at[kp]], sems[2 * slot + 1]
        ).wait()

      @pl.when(j < cpc)
      def _():
        cid = c * cpc + j
        pltpu.sync_copy(msg_h.at[pl.ds(cid * ck, ck)], val_v)
        pltpu.make_async_copy(
            val_v, accn.at[cidx_v.at[k]], sems[2 * slot]
        ).start(add=True)
        pltpu.make_async_copy(
            ev_v.at[k], accd.at[cidx_v.at[k]], sems[2 * slot + 1]
        ).start(add=True)

    def body(m, carry):
      chunk(m, 0, vala)
      chunk(m, 1, valb)
      return carry

    lax.fori_loop(0, kpt2 // 2, body, 0)
    for slot in (0, 1):
      k = kpt2 - 2 + slot

      @pl.when(k * NS + s < cpc)
      def _():
        pltpu.make_async_copy(
            vala if slot == 0 else valb,
            accn.at[cidx_v.at[k]], sems[2 * slot],
        ).wait()
        pltpu.make_async_copy(
            ev_v.at[k], accd.at[cidx_v.at[k]], sems[2 * slot + 1]
        ).wait()

    plsc.subcore_barrier()
    pltpu.sync_copy(
        accn.at[pl.ds(s * rpt, rpt)],
        num_h.at[pl.ds(c * npad + s * rpt, rpt)],
    )
    pltpu.sync_copy(
        accd.at[pl.ds(s * rpt, rpt)],
        den_h.at[pl.ds(c * npad + s * rpt, rpt)],
    )

  row2 = row_idx.reshape(n_chunks, ck)
  col2 = col_idx.reshape(n_chunks, ck)
  rat3 = rating.reshape(e_num // be, 1, be)
  pad_rows = jnp.zeros((nstage - n_user, d), jnp.float32)
  upcp = jnp.concatenate([upc, pad_rows])
  ipcp = jnp.concatenate([ipc, pad_rows])
  nums, dens = [], []
  for h in range(NSLC):
    rows_s = row2[h * ncs : (h + 1) * ncs]
    cols_s = col2[h * ncs : (h + 1) * ncs]
    ug3, ig3 = _gather(upcp, ipcp, _by_tile_g(rows_s), _by_tile_g(cols_s))
    msg, erows = edge_call(
        ug3.reshape(es, d), ig3.reshape(es, d),
        rat3[h * grid_e : (h + 1) * grid_e], rpb8, W2, b2.reshape(1, d),
        a1f, A2, ab2.reshape(1, d), A3, ab3.reshape(1, 1),
    )
    numf, denf = _scatter(
        msg, _by_tile_s(erows.reshape(ncs, ck)), _by_tile_s(cols_s), zn, zd
    )
    nums.append(numf.reshape(NC, npad, d))
    dens.append(denf.reshape(NC, npad))

  num = jnp.concatenate(nums)
  den = jnp.concatenate(dens)
  nparts = NC * NSLC

  bz = 1024
  grid_z = npad // bz
  z = pl.pallas_call(
      _final_body,
      grid=(grid_z,),
      in_specs=[
          pl.BlockSpec((nparts, bz, d), lambda i: (0, i, 0)),
          pl.BlockSpec((nparts, bz), lambda i: (0, i)),
          pl.BlockSpec((d, d), lambda i: (0, 0)),
          pl.BlockSpec((1, d), lambda i: (0, 0)),
      ],
      out_specs=pl.BlockSpec((bz, d), lambda i: (i, 0)),
      out_shape=jax.ShapeDtypeStruct((npad, d), jnp.float32),
      compiler_params=pltpu.CompilerParams(
          dimension_semantics=("arbitrary",)
      ),
  )(num, den, Ww, wb.reshape(1, d))
  return z[:n_item]
